# Initial kernel scaffold; baseline (speedup 1.0000x reference)
import jax
import jax.numpy as jnp
from jax.experimental import pallas as pl


def kernel(x, labels, origin, W1, b1, W2, b2):
    raise NotImplementedError("write your pallas kernel here")


# speedup vs baseline: 3.7261x; 3.7261x over previous
import jax
import jax.numpy as jnp
from jax.experimental import pallas as pl
from jax.experimental.pallas import tpu as pltpu

B, N, D = 64, 64, 64
H = 128


def _vertexing_kernel(x_ref, w1a_ref, w1b_ref, b1_ref, w2_ref, b2_ref,
                      lab_r_ref, lab_c_ref, org_r_ref, org_c_ref,
                      pred_ref, loss_ref, acc_s, acc_c):
    b = pl.program_id(0)
    nb = pl.num_programs(0)

    xb = x_ref[0]
    a = jnp.dot(xb, w1a_ref[...], preferred_element_type=jnp.float32)
    c = jnp.dot(xb, w1b_ref[...], preferred_element_type=jnp.float32)
    c = c + b1_ref[...]

    h = jnp.maximum(a[:, None, :] + c[None, :, :], 0.0)
    p = jnp.sum(h * w2_ref[...][None, :, :], axis=-1) + b2_ref[0, 0]
    pred_ref[0] = p

    li = lab_c_ref[0]
    lj = lab_r_ref[0]
    row = jax.lax.broadcasted_iota(jnp.int32, (N, N), 0)
    col = jax.lax.broadcasted_iota(jnp.int32, (N, N), 1)
    off = row != col
    match = (li == lj) & jnp.logical_not((li < 0) | (lj < 0)) & off
    mm = jnp.where(match, 1.0, 0.0)

    def w0(o):
        hf = ((o == 3) | (o == 4) | (o == 5)).astype(jnp.int32)
        return hf - (o == 1).astype(jnp.int32)

    wp = jnp.bitwise_and(w0(org_c_ref[0]), w0(org_r_ref[0]))
    wts = (1 + wp).astype(jnp.float32)

    bce = jnp.maximum(p, 0.0) - p * mm + jnp.log1p(jnp.exp(-jnp.abs(p)))
    contrib = jnp.sum(jnp.where(off, bce * wts, 0.0))
    cnt = jnp.sum(mm)

    @pl.when(b == 0)
    def _init():
        acc_s[0, 0] = 0.0
        acc_c[0, 0] = 0.0

    acc_s[0, 0] += contrib
    acc_c[0, 0] += cnt

    @pl.when(b == nb - 1)
    def _fin():
        loss_ref[0, 0] = acc_s[0, 0] / acc_c[0, 0]


def kernel(x, labels, origin, W1, b1, W2, b2):
    labels = labels.astype(jnp.int32)
    origin = origin.astype(jnp.int32)
    w1a = W1[:D]
    w1b = W1[D:]
    b1r = b1.reshape(1, H)
    w2r = W2.reshape(1, H)
    b2r = b2.reshape(1, 1)
    lab_r = labels.reshape(B, 1, N)
    lab_c = labels.reshape(B, N, 1)
    org_r = origin.reshape(B, 1, N)
    org_c = origin.reshape(B, N, 1)

    pred_full, loss = pl.pallas_call(
        _vertexing_kernel,
        grid=(B,),
        in_specs=[
            pl.BlockSpec((1, N, D), lambda b: (b, 0, 0)),
            pl.BlockSpec((D, H), lambda b: (0, 0)),
            pl.BlockSpec((D, H), lambda b: (0, 0)),
            pl.BlockSpec((1, H), lambda b: (0, 0)),
            pl.BlockSpec((1, H), lambda b: (0, 0)),
            pl.BlockSpec(memory_space=pltpu.SMEM),
            pl.BlockSpec((1, 1, N), lambda b: (b, 0, 0)),
            pl.BlockSpec((1, N, 1), lambda b: (b, 0, 0)),
            pl.BlockSpec((1, 1, N), lambda b: (b, 0, 0)),
            pl.BlockSpec((1, N, 1), lambda b: (b, 0, 0)),
        ],
        out_specs=[
            pl.BlockSpec((1, N, N), lambda b: (b, 0, 0)),
            pl.BlockSpec(memory_space=pltpu.SMEM),
        ],
        out_shape=[
            jax.ShapeDtypeStruct((B, N, N), jnp.float32),
            jax.ShapeDtypeStruct((1, 1), jnp.float32),
        ],
        scratch_shapes=[
            pltpu.SMEM((1, 1), jnp.float32),
            pltpu.SMEM((1, 1), jnp.float32),
        ],
    )(x, w1a, w1b, b1r, w2r, b2r, lab_r, lab_c, org_r, org_c)

    pred = (pred_full.reshape(B, N * N)[:, :-1]
            .reshape(B, N - 1, N + 1)[:, :, 1:]
            .reshape(B * N * (N - 1), 1))
    return pred, loss[0, 0]


# speedup vs baseline: 10.3670x; 2.7822x over previous
import jax
import jax.numpy as jnp
from jax.experimental import pallas as pl
from jax.experimental.pallas import tpu as pltpu

B, N, D = 64, 64, 64
H = 128


def _vertexing_kernel(x_ref, w1a_ref, w1b_ref, b1_ref, w2_ref, b2_ref,
                      lab_r_ref, lab_c_ref, org_r_ref, org_c_ref,
                      pred_ref, loss_ref, acc_s, acc_c):
    b = pl.program_id(0)
    nb = pl.num_programs(0)

    xb = x_ref[0]
    a = jnp.dot(xb, w1a_ref[...], preferred_element_type=jnp.float32)
    c = jnp.dot(xb, w1b_ref[...], preferred_element_type=jnp.float32)
    c = c + b1_ref[...]

    h = jnp.maximum(a[:, None, :] + c[None, :, :], 0.0)
    p = jnp.sum(h * w2_ref[...][None, :, :], axis=-1) + b2_ref[0, 0]
    pred_ref[0] = p
    p = pred_ref[0]

    li = lab_c_ref[0]
    lj = lab_r_ref[0]
    row = jax.lax.broadcasted_iota(jnp.int32, (N, N), 0)
    col = jax.lax.broadcasted_iota(jnp.int32, (N, N), 1)
    off = row != col
    match = (li == lj) & jnp.logical_not((li < 0) | (lj < 0)) & off
    mm = jnp.where(match, 1.0, 0.0)

    def w0(o):
        hf = ((o == 3) | (o == 4) | (o == 5)).astype(jnp.int32)
        return hf - (o == 1).astype(jnp.int32)

    wp = jnp.bitwise_and(w0(org_c_ref[0]), w0(org_r_ref[0]))
    wts = (1 + wp).astype(jnp.float32)

    bce = jnp.maximum(p, 0.0) - p * mm + jnp.log1p(jnp.exp(-jnp.abs(p)))
    contrib = jnp.sum(jnp.where(off, bce * wts, 0.0))
    cnt = jnp.sum(mm)

    @pl.when(b == 0)
    def _init():
        acc_s[0, 0] = 0.0
        acc_c[0, 0] = 0.0

    acc_s[0, 0] += contrib
    acc_c[0, 0] += cnt

    @pl.when(b == nb - 1)
    def _fin():
        loss_ref[0, 0] = acc_s[0, 0] / acc_c[0, 0]


def kernel(x, labels, origin, W1, b1, W2, b2):
    labels = labels.astype(jnp.int32)
    origin = origin.astype(jnp.int32)
    w1a = W1[:D]
    w1b = W1[D:]
    b1r = b1.reshape(1, H)
    w2r = W2.reshape(1, H)
    b2r = b2.reshape(1, 1)
    lab_r = labels.reshape(B, 1, N)
    lab_c = labels.reshape(B, N, 1)
    org_r = origin.reshape(B, 1, N)
    org_c = origin.reshape(B, N, 1)

    pred_full, loss = pl.pallas_call(
        _vertexing_kernel,
        grid=(B,),
        in_specs=[
            pl.BlockSpec((1, N, D), lambda b: (b, 0, 0)),
            pl.BlockSpec((D, H), lambda b: (0, 0)),
            pl.BlockSpec((D, H), lambda b: (0, 0)),
            pl.BlockSpec((1, H), lambda b: (0, 0)),
            pl.BlockSpec((1, H), lambda b: (0, 0)),
            pl.BlockSpec(memory_space=pltpu.SMEM),
            pl.BlockSpec((1, 1, N), lambda b: (b, 0, 0)),
            pl.BlockSpec((1, N, 1), lambda b: (b, 0, 0)),
            pl.BlockSpec((1, 1, N), lambda b: (b, 0, 0)),
            pl.BlockSpec((1, N, 1), lambda b: (b, 0, 0)),
        ],
        out_specs=[
            pl.BlockSpec((1, N, N), lambda b: (b, 0, 0)),
            pl.BlockSpec(memory_space=pltpu.SMEM),
        ],
        out_shape=[
            jax.ShapeDtypeStruct((B, N, N), jnp.float32),
            jax.ShapeDtypeStruct((1, 1), jnp.float32),
        ],
        scratch_shapes=[
            pltpu.SMEM((1, 1), jnp.float32),
            pltpu.SMEM((1, 1), jnp.float32),
        ],
    )(x, w1a, w1b, b1r, w2r, b2r, lab_r, lab_c, org_r, org_c)

    pred = (pred_full.reshape(B, N * N)[:, :-1]
            .reshape(B, N - 1, N + 1)[:, :, 1:]
            .reshape(B * N * (N - 1), 1))
    return pred, loss[0, 0]


# speedup vs baseline: 12.4102x; 1.1971x over previous
import jax
import jax.numpy as jnp
from jax.experimental import pallas as pl
from jax.experimental.pallas import tpu as pltpu

B, N, D = 64, 64, 64
H = 128
BB = 2


def _vertexing_kernel(x_ref, w1a_ref, w1b_ref, b1_ref, w2_ref, b2_ref,
                      lab_r_ref, lab_c_ref, org_r_ref, org_c_ref,
                      pred_ref, loss_ref, acc_s, acc_c):
    b = pl.program_id(0)
    nb = pl.num_programs(0)

    xb = x_ref[...].reshape(BB * N, D)
    a = jnp.dot(xb, w1a_ref[...], preferred_element_type=jnp.float32)
    c = jnp.dot(xb, w1b_ref[...], preferred_element_type=jnp.float32)
    a = a.reshape(BB, N, 1, H)
    c = (c + b1_ref[...]).reshape(BB, 1, N, H)

    h = jnp.maximum(a + c, 0.0)
    p = jnp.sum(h * w2_ref[...], axis=-1) + b2_ref[0, 0]
    pred_ref[...] = p
    p = pred_ref[...]

    li = lab_c_ref[...]
    lj = lab_r_ref[...]
    row = jax.lax.broadcasted_iota(jnp.int32, (BB, N, N), 1)
    col = jax.lax.broadcasted_iota(jnp.int32, (BB, N, N), 2)
    off = row != col
    match = (li == lj) & jnp.logical_not((li < 0) | (lj < 0)) & off
    mm = jnp.where(match, 1.0, 0.0)

    def w0(o):
        hf = ((o == 3) | (o == 4) | (o == 5)).astype(jnp.int32)
        return hf - (o == 1).astype(jnp.int32)

    wp = jnp.bitwise_and(w0(org_c_ref[...]), w0(org_r_ref[...]))
    wts = (1 + wp).astype(jnp.float32)

    bce = jnp.maximum(p, 0.0) - p * mm + jnp.log1p(jnp.exp(-jnp.abs(p)))
    contrib = jnp.sum(jnp.where(off, bce * wts, 0.0))
    cnt = jnp.sum(mm)

    @pl.when(b == 0)
    def _init():
        acc_s[0, 0] = 0.0
        acc_c[0, 0] = 0.0

    acc_s[0, 0] += contrib
    acc_c[0, 0] += cnt

    @pl.when(b == nb - 1)
    def _fin():
        loss_ref[0, 0] = acc_s[0, 0] / acc_c[0, 0]


def kernel(x, labels, origin, W1, b1, W2, b2):
    labels = labels.astype(jnp.int32)
    origin = origin.astype(jnp.int32)
    w1a = W1[:D]
    w1b = W1[D:]
    b1r = b1.reshape(1, H)
    w2r = W2.reshape(1, H)
    b2r = b2.reshape(1, 1)
    lab_r = labels.reshape(B, 1, N)
    lab_c = labels.reshape(B, N, 1)
    org_r = origin.reshape(B, 1, N)
    org_c = origin.reshape(B, N, 1)

    pred_full, loss = pl.pallas_call(
        _vertexing_kernel,
        grid=(B // BB,),
        in_specs=[
            pl.BlockSpec((BB, N, D), lambda b: (b, 0, 0)),
            pl.BlockSpec((D, H), lambda b: (0, 0)),
            pl.BlockSpec((D, H), lambda b: (0, 0)),
            pl.BlockSpec((1, H), lambda b: (0, 0)),
            pl.BlockSpec((1, H), lambda b: (0, 0)),
            pl.BlockSpec(memory_space=pltpu.SMEM),
            pl.BlockSpec((BB, 1, N), lambda b: (b, 0, 0)),
            pl.BlockSpec((BB, N, 1), lambda b: (b, 0, 0)),
            pl.BlockSpec((BB, 1, N), lambda b: (b, 0, 0)),
            pl.BlockSpec((BB, N, 1), lambda b: (b, 0, 0)),
        ],
        out_specs=[
            pl.BlockSpec((BB, N, N), lambda b: (b, 0, 0)),
            pl.BlockSpec(memory_space=pltpu.SMEM),
        ],
        out_shape=[
            jax.ShapeDtypeStruct((B, N, N), jnp.float32),
            jax.ShapeDtypeStruct((1, 1), jnp.float32),
        ],
        scratch_shapes=[
            pltpu.SMEM((1, 1), jnp.float32),
            pltpu.SMEM((1, 1), jnp.float32),
        ],
    )(x, w1a, w1b, b1r, w2r, b2r, lab_r, lab_c, org_r, org_c)

    pred = (pred_full.reshape(B, N * N)[:, :-1]
            .reshape(B, N - 1, N + 1)[:, :, 1:]
            .reshape(B * N * (N - 1), 1))
    return pred, loss[0, 0]


# speedup vs baseline: 13.6330x; 1.0985x over previous
import jax
import jax.numpy as jnp
from jax.experimental import pallas as pl
from jax.experimental.pallas import tpu as pltpu

B, N, D = 64, 64, 64
H = 128
BB = 4


def _vertexing_kernel(x_ref, w1a_ref, w1b_ref, b1_ref, w2_ref, b2_ref,
                      lab_r_ref, lab_c_ref, org_r_ref, org_c_ref,
                      pred_ref, loss_ref, acc_s, acc_c):
    b = pl.program_id(0)
    nb = pl.num_programs(0)

    xb = x_ref[...].reshape(BB * N, D)
    a = jnp.dot(xb, w1a_ref[...], preferred_element_type=jnp.float32)
    c = jnp.dot(xb, w1b_ref[...], preferred_element_type=jnp.float32)
    a = a.reshape(BB, N, 1, H)
    c = (c + b1_ref[...]).reshape(BB, 1, N, H)

    h = jnp.maximum(a + c, 0.0)
    p = jnp.sum(h * w2_ref[...], axis=-1) + b2_ref[0, 0]
    pred_ref[...] = p
    p = pred_ref[...]

    li = lab_c_ref[...]
    lj = lab_r_ref[...]
    row = jax.lax.broadcasted_iota(jnp.int32, (BB, N, N), 1)
    col = jax.lax.broadcasted_iota(jnp.int32, (BB, N, N), 2)
    off = row != col
    match = (li == lj) & jnp.logical_not((li < 0) | (lj < 0)) & off
    mm = jnp.where(match, 1.0, 0.0)

    def w0(o):
        hf = ((o == 3) | (o == 4) | (o == 5)).astype(jnp.int32)
        return hf - (o == 1).astype(jnp.int32)

    wp = jnp.bitwise_and(w0(org_c_ref[...]), w0(org_r_ref[...]))
    wts = (1 + wp).astype(jnp.float32)

    bce = jnp.maximum(p, 0.0) - p * mm + jnp.log1p(jnp.exp(-jnp.abs(p)))
    contrib = jnp.sum(jnp.where(off, bce * wts, 0.0))
    cnt = jnp.sum(mm)

    @pl.when(b == 0)
    def _init():
        acc_s[0, 0] = 0.0
        acc_c[0, 0] = 0.0

    acc_s[0, 0] += contrib
    acc_c[0, 0] += cnt

    @pl.when(b == nb - 1)
    def _fin():
        loss_ref[0, 0] = acc_s[0, 0] / acc_c[0, 0]


def kernel(x, labels, origin, W1, b1, W2, b2):
    labels = labels.astype(jnp.int32)
    origin = origin.astype(jnp.int32)
    w1a = W1[:D]
    w1b = W1[D:]
    b1r = b1.reshape(1, H)
    w2r = W2.reshape(1, H)
    b2r = b2.reshape(1, 1)
    lab_r = labels.reshape(B, 1, N)
    lab_c = labels.reshape(B, N, 1)
    org_r = origin.reshape(B, 1, N)
    org_c = origin.reshape(B, N, 1)

    pred_full, loss = pl.pallas_call(
        _vertexing_kernel,
        grid=(B // BB,),
        in_specs=[
            pl.BlockSpec((BB, N, D), lambda b: (b, 0, 0)),
            pl.BlockSpec((D, H), lambda b: (0, 0)),
            pl.BlockSpec((D, H), lambda b: (0, 0)),
            pl.BlockSpec((1, H), lambda b: (0, 0)),
            pl.BlockSpec((1, H), lambda b: (0, 0)),
            pl.BlockSpec(memory_space=pltpu.SMEM),
            pl.BlockSpec((BB, 1, N), lambda b: (b, 0, 0)),
            pl.BlockSpec((BB, N, 1), lambda b: (b, 0, 0)),
            pl.BlockSpec((BB, 1, N), lambda b: (b, 0, 0)),
            pl.BlockSpec((BB, N, 1), lambda b: (b, 0, 0)),
        ],
        out_specs=[
            pl.BlockSpec((BB, N, N), lambda b: (b, 0, 0)),
            pl.BlockSpec(memory_space=pltpu.SMEM),
        ],
        out_shape=[
            jax.ShapeDtypeStruct((B, N, N), jnp.float32),
            jax.ShapeDtypeStruct((1, 1), jnp.float32),
        ],
        scratch_shapes=[
            pltpu.SMEM((1, 1), jnp.float32),
            pltpu.SMEM((1, 1), jnp.float32),
        ],
    )(x, w1a, w1b, b1r, w2r, b2r, lab_r, lab_c, org_r, org_c)

    pred = (pred_full.reshape(B, N * N)[:, :-1]
            .reshape(B, N - 1, N + 1)[:, :, 1:]
            .reshape(B * N * (N - 1), 1))
    return pred, loss[0, 0]


# speedup vs baseline: 14.1798x; 1.0401x over previous
import jax
import jax.numpy as jnp
from jax.experimental import pallas as pl
from jax.experimental.pallas import tpu as pltpu

B, N, D = 64, 64, 64
H = 128
BB = 8


def _vertexing_kernel(x_ref, w1a_ref, w1b_ref, b1_ref, w2_ref, b2_ref,
                      lab_r_ref, lab_c_ref, org_r_ref, org_c_ref,
                      pred_ref, loss_ref, acc_s, acc_c):
    b = pl.program_id(0)
    nb = pl.num_programs(0)

    xb = x_ref[...].reshape(BB * N, D)
    a = jnp.dot(xb, w1a_ref[...], preferred_element_type=jnp.float32)
    c = jnp.dot(xb, w1b_ref[...], preferred_element_type=jnp.float32)
    a = a.reshape(BB, N, 1, H)
    c = (c + b1_ref[...]).reshape(BB, 1, N, H)

    h = jnp.maximum(a + c, 0.0)
    p = jnp.sum(h * w2_ref[...], axis=-1) + b2_ref[0, 0]
    pred_ref[...] = p
    p = pred_ref[...]

    li = lab_c_ref[...]
    lj = lab_r_ref[...]
    row = jax.lax.broadcasted_iota(jnp.int32, (BB, N, N), 1)
    col = jax.lax.broadcasted_iota(jnp.int32, (BB, N, N), 2)
    off = row != col
    match = (li == lj) & jnp.logical_not((li < 0) | (lj < 0)) & off
    mm = jnp.where(match, 1.0, 0.0)

    def w0(o):
        hf = ((o == 3) | (o == 4) | (o == 5)).astype(jnp.int32)
        return hf - (o == 1).astype(jnp.int32)

    wp = jnp.bitwise_and(w0(org_c_ref[...]), w0(org_r_ref[...]))
    wts = (1 + wp).astype(jnp.float32)

    bce = jnp.maximum(p, 0.0) - p * mm + jnp.log1p(jnp.exp(-jnp.abs(p)))
    contrib = jnp.sum(jnp.where(off, bce * wts, 0.0))
    cnt = jnp.sum(mm)

    @pl.when(b == 0)
    def _init():
        acc_s[0, 0] = 0.0
        acc_c[0, 0] = 0.0

    acc_s[0, 0] += contrib
    acc_c[0, 0] += cnt

    @pl.when(b == nb - 1)
    def _fin():
        loss_ref[0, 0] = acc_s[0, 0] / acc_c[0, 0]


def kernel(x, labels, origin, W1, b1, W2, b2):
    labels = labels.astype(jnp.int32)
    origin = origin.astype(jnp.int32)
    w1a = W1[:D]
    w1b = W1[D:]
    b1r = b1.reshape(1, H)
    w2r = W2.reshape(1, H)
    b2r = b2.reshape(1, 1)
    lab_r = labels.reshape(B, 1, N)
    lab_c = labels.reshape(B, N, 1)
    org_r = origin.reshape(B, 1, N)
    org_c = origin.reshape(B, N, 1)

    pred_full, loss = pl.pallas_call(
        _vertexing_kernel,
        grid=(B // BB,),
        in_specs=[
            pl.BlockSpec((BB, N, D), lambda b: (b, 0, 0)),
            pl.BlockSpec((D, H), lambda b: (0, 0)),
            pl.BlockSpec((D, H), lambda b: (0, 0)),
            pl.BlockSpec((1, H), lambda b: (0, 0)),
            pl.BlockSpec((1, H), lambda b: (0, 0)),
            pl.BlockSpec(memory_space=pltpu.SMEM),
            pl.BlockSpec((BB, 1, N), lambda b: (b, 0, 0)),
            pl.BlockSpec((BB, N, 1), lambda b: (b, 0, 0)),
            pl.BlockSpec((BB, 1, N), lambda b: (b, 0, 0)),
            pl.BlockSpec((BB, N, 1), lambda b: (b, 0, 0)),
        ],
        out_specs=[
            pl.BlockSpec((BB, N, N), lambda b: (b, 0, 0)),
            pl.BlockSpec(memory_space=pltpu.SMEM),
        ],
        out_shape=[
            jax.ShapeDtypeStruct((B, N, N), jnp.float32),
            jax.ShapeDtypeStruct((1, 1), jnp.float32),
        ],
        scratch_shapes=[
            pltpu.SMEM((1, 1), jnp.float32),
            pltpu.SMEM((1, 1), jnp.float32),
        ],
    )(x, w1a, w1b, b1r, w2r, b2r, lab_r, lab_c, org_r, org_c)

    pred = (pred_full.reshape(B, N * N)[:, :-1]
            .reshape(B, N - 1, N + 1)[:, :, 1:]
            .reshape(B * N * (N - 1), 1))
    return pred, loss[0, 0]


# speedup vs baseline: 14.2853x; 1.0074x over previous
import jax
import jax.numpy as jnp
from jax.experimental import pallas as pl
from jax.experimental.pallas import tpu as pltpu

B, N, D = 64, 64, 64
H = 128
BB = 16


def _vertexing_kernel(x_ref, w1a_ref, w1b_ref, b1_ref, w2_ref, b2_ref,
                      lab_r_ref, lab_c_ref, org_r_ref, org_c_ref,
                      pred_ref, loss_ref, acc_s, acc_c):
    b = pl.program_id(0)
    nb = pl.num_programs(0)

    xb = x_ref[...].reshape(BB * N, D)
    a = jnp.dot(xb, w1a_ref[...], preferred_element_type=jnp.float32)
    c = jnp.dot(xb, w1b_ref[...], preferred_element_type=jnp.float32)
    a = a.reshape(BB, N, 1, H)
    c = (c + b1_ref[...]).reshape(BB, 1, N, H)

    h = jnp.maximum(a + c, 0.0)
    p = jnp.sum(h * w2_ref[...], axis=-1) + b2_ref[0, 0]
    pred_ref[...] = p
    p = pred_ref[...]

    li = lab_c_ref[...]
    lj = lab_r_ref[...]
    row = jax.lax.broadcasted_iota(jnp.int32, (BB, N, N), 1)
    col = jax.lax.broadcasted_iota(jnp.int32, (BB, N, N), 2)
    off = row != col
    match = (li == lj) & jnp.logical_not((li < 0) | (lj < 0)) & off
    mm = jnp.where(match, 1.0, 0.0)

    def w0(o):
        hf = ((o == 3) | (o == 4) | (o == 5)).astype(jnp.int32)
        return hf - (o == 1).astype(jnp.int32)

    wp = jnp.bitwise_and(w0(org_c_ref[...]), w0(org_r_ref[...]))
    wts = (1 + wp).astype(jnp.float32)

    bce = jnp.maximum(p, 0.0) - p * mm + jnp.log1p(jnp.exp(-jnp.abs(p)))
    contrib = jnp.sum(jnp.where(off, bce * wts, 0.0))
    cnt = jnp.sum(mm)

    @pl.when(b == 0)
    def _init():
        acc_s[0, 0] = 0.0
        acc_c[0, 0] = 0.0

    acc_s[0, 0] += contrib
    acc_c[0, 0] += cnt

    @pl.when(b == nb - 1)
    def _fin():
        loss_ref[0, 0] = acc_s[0, 0] / acc_c[0, 0]


def kernel(x, labels, origin, W1, b1, W2, b2):
    labels = labels.astype(jnp.int32)
    origin = origin.astype(jnp.int32)
    w1a = W1[:D]
    w1b = W1[D:]
    b1r = b1.reshape(1, H)
    w2r = W2.reshape(1, H)
    b2r = b2.reshape(1, 1)
    lab_r = labels.reshape(B, 1, N)
    lab_c = labels.reshape(B, N, 1)
    org_r = origin.reshape(B, 1, N)
    org_c = origin.reshape(B, N, 1)

    pred_full, loss = pl.pallas_call(
        _vertexing_kernel,
        grid=(B // BB,),
        in_specs=[
            pl.BlockSpec((BB, N, D), lambda b: (b, 0, 0)),
            pl.BlockSpec((D, H), lambda b: (0, 0)),
            pl.BlockSpec((D, H), lambda b: (0, 0)),
            pl.BlockSpec((1, H), lambda b: (0, 0)),
            pl.BlockSpec((1, H), lambda b: (0, 0)),
            pl.BlockSpec(memory_space=pltpu.SMEM),
            pl.BlockSpec((BB, 1, N), lambda b: (b, 0, 0)),
            pl.BlockSpec((BB, N, 1), lambda b: (b, 0, 0)),
            pl.BlockSpec((BB, 1, N), lambda b: (b, 0, 0)),
            pl.BlockSpec((BB, N, 1), lambda b: (b, 0, 0)),
        ],
        out_specs=[
            pl.BlockSpec((BB, N, N), lambda b: (b, 0, 0)),
            pl.BlockSpec(memory_space=pltpu.SMEM),
        ],
        out_shape=[
            jax.ShapeDtypeStruct((B, N, N), jnp.float32),
            jax.ShapeDtypeStruct((1, 1), jnp.float32),
        ],
        scratch_shapes=[
            pltpu.SMEM((1, 1), jnp.float32),
            pltpu.SMEM((1, 1), jnp.float32),
        ],
    )(x, w1a, w1b, b1r, w2r, b2r, lab_r, lab_c, org_r, org_c)

    pred = (pred_full.reshape(B, N * N)[:, :-1]
            .reshape(B, N - 1, N + 1)[:, :, 1:]
            .reshape(B * N * (N - 1), 1))
    return pred, loss[0, 0]


# speedup vs baseline: 14.3968x; 1.0078x over previous
import jax
import jax.numpy as jnp
from jax.experimental import pallas as pl
from jax.experimental.pallas import tpu as pltpu

B, N, D = 64, 64, 64
H = 128
BB = 16


def _vertexing_kernel(x_ref, w1a_ref, w1b_ref, b1_ref, w2_ref, b2_ref,
                      lab_r_ref, lab_c_ref, org_r_ref, org_c_ref,
                      pred_ref, loss_ref, acc_s, acc_c):
    b = pl.program_id(0)
    nb = pl.num_programs(0)

    xb = x_ref[...].reshape(BB * N, D)
    a = jnp.dot(xb, w1a_ref[...], preferred_element_type=jnp.float32)
    c = jnp.dot(xb, w1b_ref[...], preferred_element_type=jnp.float32)
    a = a.reshape(BB, N, 1, H)
    c = (c + b1_ref[...]).reshape(BB, 1, N, H)

    h = jnp.maximum(a + c, 0.0)
    p = jnp.sum(h * w2_ref[...], axis=-1) + b2_ref[0, 0]
    pred_ref[...] = p
    p = pred_ref[...]

    li = lab_c_ref[...]
    lj = lab_r_ref[...]
    row = jax.lax.broadcasted_iota(jnp.int32, (BB, N, N), 1)
    col = jax.lax.broadcasted_iota(jnp.int32, (BB, N, N), 2)
    off = row != col
    match = (li == lj) & jnp.logical_not((li < 0) | (lj < 0)) & off
    mm = jnp.where(match, 1.0, 0.0)

    def w0(o):
        hf = ((o == 3) | (o == 4) | (o == 5)).astype(jnp.int32)
        return hf - (o == 1).astype(jnp.int32)

    wp = jnp.bitwise_and(w0(org_c_ref[...]), w0(org_r_ref[...]))
    wts = (1 + wp).astype(jnp.float32)

    bce = jnp.maximum(p, 0.0) - p * mm + jnp.log1p(jnp.exp(-jnp.abs(p)))
    contrib = jnp.sum(jnp.where(off, bce * wts, 0.0).reshape(-1, 8, N), axis=0)
    cnt = jnp.sum(mm.reshape(-1, 8, N), axis=0)

    @pl.when(b == 0)
    def _init():
        acc_s[...] = jnp.zeros((8, N), jnp.float32)
        acc_c[...] = jnp.zeros((8, N), jnp.float32)

    acc_s[...] += contrib
    acc_c[...] += cnt

    @pl.when(b == nb - 1)
    def _fin():
        loss_ref[0, 0] = jnp.sum(acc_s[...]) / jnp.sum(acc_c[...])


def kernel(x, labels, origin, W1, b1, W2, b2):
    labels = labels.astype(jnp.int32)
    origin = origin.astype(jnp.int32)
    w1a = W1[:D]
    w1b = W1[D:]
    b1r = b1.reshape(1, H)
    w2r = W2.reshape(1, H)
    b2r = b2.reshape(1, 1)
    lab_r = labels.reshape(B, 1, N)
    lab_c = labels.reshape(B, N, 1)
    org_r = origin.reshape(B, 1, N)
    org_c = origin.reshape(B, N, 1)

    pred_full, loss = pl.pallas_call(
        _vertexing_kernel,
        grid=(B // BB,),
        in_specs=[
            pl.BlockSpec((BB, N, D), lambda b: (b, 0, 0)),
            pl.BlockSpec((D, H), lambda b: (0, 0)),
            pl.BlockSpec((D, H), lambda b: (0, 0)),
            pl.BlockSpec((1, H), lambda b: (0, 0)),
            pl.BlockSpec((1, H), lambda b: (0, 0)),
            pl.BlockSpec(memory_space=pltpu.SMEM),
            pl.BlockSpec((BB, 1, N), lambda b: (b, 0, 0)),
            pl.BlockSpec((BB, N, 1), lambda b: (b, 0, 0)),
            pl.BlockSpec((BB, 1, N), lambda b: (b, 0, 0)),
            pl.BlockSpec((BB, N, 1), lambda b: (b, 0, 0)),
        ],
        out_specs=[
            pl.BlockSpec((BB, N, N), lambda b: (b, 0, 0)),
            pl.BlockSpec(memory_space=pltpu.SMEM),
        ],
        out_shape=[
            jax.ShapeDtypeStruct((B, N, N), jnp.float32),
            jax.ShapeDtypeStruct((1, 1), jnp.float32),
        ],
        scratch_shapes=[
            pltpu.VMEM((8, N), jnp.float32),
            pltpu.VMEM((8, N), jnp.float32),
        ],
    )(x, w1a, w1b, b1r, w2r, b2r, lab_r, lab_c, org_r, org_c)

    pred = (pred_full.reshape(B, N * N)[:, :-1]
            .reshape(B, N - 1, N + 1)[:, :, 1:]
            .reshape(B * N * (N - 1), 1))
    return pred, loss[0, 0]
